# Initial kernel scaffold; baseline (speedup 1.0000x reference)
import jax
import jax.numpy as jnp
from jax.experimental import pallas as pl


def kernel(box_scores, box_preds):
    raise NotImplementedError("write your pallas kernel here")


# speedup vs baseline: 14.3783x; 14.3783x over previous
import jax
import jax.numpy as jnp
from jax.experimental import pallas as pl
from jax.experimental.pallas import tpu as pltpu

_N = 20000
_PRE = 4096
_POST = 500
_NMS_THRESH = 0.1
_SCORE_THRESH = 0.1
_R = 32
_C = 128


def _nms_body(colaux_ref, valid_ref, keep_ref):
    keep_ref[...] = valid_ref[...]
    gid = (jax.lax.broadcasted_iota(jnp.int32, (_R, _C), 0) * _C
           + jax.lax.broadcasted_iota(jnp.int32, (_R, _C), 1))
    lane = jax.lax.broadcasted_iota(jnp.int32, (1, _C), 1)
    mnx = colaux_ref[0]
    mny = colaux_ref[1]
    mnz = colaux_ref[2]
    mxx = colaux_ref[3]
    mxy = colaux_ref[4]
    mxz = colaux_ref[5]
    vol = colaux_ref[6]

    def step(i, carry):
        r = i // _C
        lmask = lane == (i % _C)

        def pick(d):
            return jnp.sum(jnp.where(lmask, colaux_ref[d, pl.ds(r, 1), :], 0.0))

        ki = jnp.sum(jnp.where(lmask, keep_ref[pl.ds(r, 1), :], 0.0))
        b_mnx = pick(0)
        b_mny = pick(1)
        b_mnz = pick(2)
        b_mxx = pick(3)
        b_mxy = pick(4)
        b_mxz = pick(5)
        b_vol = pick(6)
        ix = jnp.maximum(jnp.minimum(b_mxx, mxx) - jnp.maximum(b_mnx, mnx), 0.0)
        iy = jnp.maximum(jnp.minimum(b_mxy, mxy) - jnp.maximum(b_mny, mny), 0.0)
        iz = jnp.maximum(jnp.minimum(b_mxz, mxz) - jnp.maximum(b_mnz, mnz), 0.0)
        inter = ix * iy * iz
        union = jnp.maximum(b_vol + vol - inter, 1e-6)
        sup = (inter > _NMS_THRESH * union) & (gid > i) & (ki > 0.0)
        keep_ref[...] = jnp.where(sup, 0.0, keep_ref[...])
        return carry

    jax.lax.fori_loop(0, _PRE, step, 0)


def kernel(box_scores, box_preds):
    masked = jnp.where(box_scores > _SCORE_THRESH, box_scores, -1.0)
    _, idx = jax.lax.top_k(masked, _PRE)
    s = box_scores[idx]
    b = box_preds[idx]
    c = b[:, :3]
    d = jnp.abs(b[:, 3:6])
    mn = c - d * 0.5
    mx = c + d * 0.5
    vol = (d[:, 0] * d[:, 1] * d[:, 2])[:, None]
    colaux = jnp.concatenate([mn.T, mx.T, vol.T], axis=0).reshape(7, _R, _C)
    valid = (s > _SCORE_THRESH).astype(jnp.float32).reshape(_R, _C)
    keep_f = pl.pallas_call(
        _nms_body,
        out_shape=jax.ShapeDtypeStruct((_R, _C), jnp.float32),
    )(colaux, valid)
    keep = keep_f.reshape(_PRE) > 0.5
    kept_masked = jnp.where(keep, s, -1.0)
    _, order = jax.lax.top_k(kept_masked, _POST)
    sel_valid = keep[order]
    sel_scores = jnp.where(sel_valid, s[order], 0.0)
    sel_idx = jnp.where(sel_valid, idx[order], -1)
    return sel_scores, sel_idx


# speedup vs baseline: 36.7739x; 2.5576x over previous
import jax
import jax.numpy as jnp
from jax.experimental import pallas as pl
from jax.experimental.pallas import tpu as pltpu

_N = 20000
_PRE = 4096
_POST = 500
_NMS_THRESH = 0.1
_SCORE_THRESH = 0.1
_R = 32
_C = 128


def _sup_tile(rowaux_ref, colaux_ref, bi, cj):
    r0 = bi * _C

    def ra(d):
        return rowaux_ref[d, pl.ds(r0, _C), :]

    def ca(d):
        return colaux_ref[d, pl.ds(cj, 1), :]

    ix = jnp.maximum(jnp.minimum(ra(3), ca(3)) - jnp.maximum(ra(0), ca(0)), 0.0)
    iy = jnp.maximum(jnp.minimum(ra(4), ca(4)) - jnp.maximum(ra(1), ca(1)), 0.0)
    iz = jnp.maximum(jnp.minimum(ra(5), ca(5)) - jnp.maximum(ra(2), ca(2)), 0.0)
    inter = ix * iy * iz
    union = jnp.maximum(ra(6) + ca(6) - inter, 1e-6)
    return (inter > _NMS_THRESH * union).astype(jnp.float32)


def _nms_body(colaux_ref, rowaux_ref, valid_ref, keep_ref):
    keep_ref[...] = valid_ref[...]
    sub = jax.lax.broadcasted_iota(jnp.int32, (_C, _C), 0)
    lanesq = jax.lax.broadcasted_iota(jnp.int32, (_C, _C), 1)

    def block(bi, carry):
        s_bb = _sup_tile(rowaux_ref, colaux_ref, bi, bi)
        s_bb = jnp.where(lanesq > sub, s_bb, 0.0)
        kb = keep_ref[pl.ds(bi, 1), :]

        def w_cond(c):
            return c[1]

        def w_body(c):
            x, _ = c
            sup = jnp.dot(x, s_bb, preferred_element_type=jnp.float32)
            nx = jnp.where(sup > 0.0, 0.0, kb)
            return nx, jnp.any(nx != x)

        x, _ = jax.lax.while_loop(w_cond, w_body, (kb, True))
        keep_ref[pl.ds(bi, 1), :] = x

        def chunk(cj, carry2):
            s_bc = _sup_tile(rowaux_ref, colaux_ref, bi, cj)
            sup = jnp.dot(x, s_bc, preferred_element_type=jnp.float32)
            krow = keep_ref[pl.ds(cj, 1), :]
            keep_ref[pl.ds(cj, 1), :] = jnp.where(sup > 0.0, 0.0, krow)
            return carry2

        jax.lax.fori_loop(bi + 1, _R, chunk, 0)
        return carry

    jax.lax.fori_loop(0, _R, block, 0)


def kernel(box_scores, box_preds):
    masked = jnp.where(box_scores > _SCORE_THRESH, box_scores, -1.0)
    _, idx = jax.lax.top_k(masked, _PRE)
    s = box_scores[idx]
    b = box_preds[idx]
    c = b[:, :3]
    d = jnp.abs(b[:, 3:6])
    mn = c - d * 0.5
    mx = c + d * 0.5
    vol = (d[:, 0] * d[:, 1] * d[:, 2])[:, None]
    cat = jnp.concatenate([mn, mx, vol], axis=1).T
    colaux = cat.reshape(7, _R, _C)
    rowaux = cat.reshape(7, _PRE, 1)
    valid = (s > _SCORE_THRESH).astype(jnp.float32).reshape(_R, _C)
    keep_f = pl.pallas_call(
        _nms_body,
        out_shape=jax.ShapeDtypeStruct((_R, _C), jnp.float32),
    )(colaux, rowaux, valid)
    keep = keep_f.reshape(_PRE) > 0.5
    kept_masked = jnp.where(keep, s, -1.0)
    _, order = jax.lax.top_k(kept_masked, _POST)
    sel_valid = keep[order]
    sel_scores = jnp.where(sel_valid, s[order], 0.0)
    sel_idx = jnp.where(sel_valid, idx[order], -1)
    return sel_scores, sel_idx


# speedup vs baseline: 49.4065x; 1.3435x over previous
import jax
import jax.numpy as jnp
from jax.experimental import pallas as pl
from jax.experimental.pallas import tpu as pltpu

_N = 20000
_PRE = 4096
_POST = 500
_NMS_THRESH = 0.1
_SCORE_THRESH = 0.1
_R = 32
_C = 128


def _sup_tile(rb_ref, colaux_ref, cj):
    def ra(d):
        return rb_ref[d]

    def ca(d):
        return colaux_ref[d, pl.ds(cj, 1), :]

    ix = jnp.maximum(jnp.minimum(ra(3), ca(3)) - jnp.maximum(ra(0), ca(0)), 0.0)
    iy = jnp.maximum(jnp.minimum(ra(4), ca(4)) - jnp.maximum(ra(1), ca(1)), 0.0)
    iz = jnp.maximum(jnp.minimum(ra(5), ca(5)) - jnp.maximum(ra(2), ca(2)), 0.0)
    inter = ix * iy * iz
    union = jnp.maximum(ra(6) + ca(6) - inter, 1e-6)
    return (inter > _NMS_THRESH * union).astype(jnp.float32)


def _nms_body(colaux_ref, rowaux_ref, valid_ref, keep_ref, rb_ref):
    keep_ref[...] = valid_ref[...]
    sub = jax.lax.broadcasted_iota(jnp.int32, (_C, _C), 0)
    lanesq = jax.lax.broadcasted_iota(jnp.int32, (_C, _C), 1)

    def block(bi, carry):
        r0 = bi * _C
        for d in range(7):
            rb_ref[d] = jnp.broadcast_to(
                rowaux_ref[d, pl.ds(r0, _C), :], (_C, _C))
        s_bb = _sup_tile(rb_ref, colaux_ref, bi)
        s_bb = jnp.where(lanesq > sub, s_bb, 0.0)
        kb = keep_ref[pl.ds(bi, 1), :]

        def w_cond(c):
            return c[1]

        def w_body(c):
            x, _ = c
            sup = jnp.dot(x, s_bb, preferred_element_type=jnp.float32)
            nx = jnp.where(sup > 0.0, 0.0, kb)
            return nx, jnp.any(nx != x)

        x, _ = jax.lax.while_loop(w_cond, w_body, (kb, True))
        keep_ref[pl.ds(bi, 1), :] = x

        def chunk(cj, carry2):
            s_bc = _sup_tile(rb_ref, colaux_ref, cj)
            sup = jnp.dot(x, s_bc, preferred_element_type=jnp.float32)
            krow = keep_ref[pl.ds(cj, 1), :]
            keep_ref[pl.ds(cj, 1), :] = jnp.where(sup > 0.0, 0.0, krow)
            return carry2

        jax.lax.fori_loop(bi + 1, _R, chunk, 0)
        return carry

    jax.lax.fori_loop(0, _R, block, 0)


def kernel(box_scores, box_preds):
    masked = jnp.where(box_scores > _SCORE_THRESH, box_scores, -1.0)
    _, idx = jax.lax.top_k(masked, _PRE)
    s = box_scores[idx]
    b = box_preds[idx]
    c = b[:, :3]
    d = jnp.abs(b[:, 3:6])
    mn = c - d * 0.5
    mx = c + d * 0.5
    vol = (d[:, 0] * d[:, 1] * d[:, 2])[:, None]
    cat = jnp.concatenate([mn, mx, vol], axis=1).T
    colaux = cat.reshape(7, _R, _C)
    rowaux = cat.reshape(7, _PRE, 1)
    valid = (s > _SCORE_THRESH).astype(jnp.float32).reshape(_R, _C)
    keep_f = pl.pallas_call(
        _nms_body,
        out_shape=jax.ShapeDtypeStruct((_R, _C), jnp.float32),
        scratch_shapes=[pltpu.VMEM((7, _C, _C), jnp.float32)],
    )(colaux, rowaux, valid)
    keep = keep_f.reshape(_PRE) > 0.5
    kept_masked = jnp.where(keep, s, -1.0)
    _, order = jax.lax.top_k(kept_masked, _POST)
    sel_valid = keep[order]
    sel_scores = jnp.where(sel_valid, s[order], 0.0)
    sel_idx = jnp.where(sel_valid, idx[order], -1)
    return sel_scores, sel_idx


# speedup vs baseline: 63.6044x; 1.2874x over previous
import jax
import jax.numpy as jnp
from jax.experimental import pallas as pl
from jax.experimental.pallas import tpu as pltpu

_N = 20000
_PRE = 4096
_POST = 500
_NMS_THRESH = 0.1
_SCORE_THRESH = 0.1
_R = 32
_C = 128


def _sup_tile(rb_ref, colaux_ref, cj):
    def ra(d):
        return rb_ref[d]

    def ca(d):
        return colaux_ref[d, pl.ds(cj, 1), :]

    ix = jnp.maximum(jnp.minimum(ra(3), ca(3)) - jnp.maximum(ra(0), ca(0)), 0.0)
    iy = jnp.maximum(jnp.minimum(ra(4), ca(4)) - jnp.maximum(ra(1), ca(1)), 0.0)
    iz = jnp.maximum(jnp.minimum(ra(5), ca(5)) - jnp.maximum(ra(2), ca(2)), 0.0)
    inter = ix * iy * iz
    union = jnp.maximum(ra(6) + ca(6) - inter, 1e-6)
    return (inter > _NMS_THRESH * union).astype(jnp.float32)


def _nms_body(colaux_ref, rowaux_ref, valid_ref, keep_ref, rb_ref):
    keep_ref[...] = valid_ref[...]
    sub = jax.lax.broadcasted_iota(jnp.int32, (_C, _C), 0)
    lanesq = jax.lax.broadcasted_iota(jnp.int32, (_C, _C), 1)

    def block(bi, carry):
        r0 = bi * _C
        for d in range(7):
            rb_ref[d] = jnp.broadcast_to(
                rowaux_ref[d, pl.ds(r0, _C), :], (_C, _C))
        s_bb = _sup_tile(rb_ref, colaux_ref, bi)
        s_bb = jnp.where(lanesq > sub, s_bb, 0.0)
        kb = keep_ref[pl.ds(bi, 1), :]

        def w_cond(c):
            return c[1]

        def w_body(c):
            x, _ = c
            sup = jnp.dot(x, s_bb, preferred_element_type=jnp.float32)
            nx = jnp.where(sup > 0.0, 0.0, kb)
            return nx, jnp.any(nx != x)

        x, _ = jax.lax.while_loop(w_cond, w_body, (kb, True))
        keep_ref[pl.ds(bi, 1), :] = x

        def chunk4(q, carry2):
            for t in range(4):
                cj = q * 4 + t
                s_bc = _sup_tile(rb_ref, colaux_ref, cj)
                sup = jnp.dot(x, s_bc, preferred_element_type=jnp.float32)
                live = (cj > bi).astype(jnp.float32)
                krow = keep_ref[pl.ds(cj, 1), :]
                keep_ref[pl.ds(cj, 1), :] = jnp.where(
                    sup * live > 0.0, 0.0, krow)
            return carry2

        jax.lax.fori_loop((bi + 1) // 4, _R // 4, chunk4, 0)
        return carry

    jax.lax.fori_loop(0, _R, block, 0)


def kernel(box_scores, box_preds):
    masked = jnp.where(box_scores > _SCORE_THRESH, box_scores, -1.0)
    _, idx = jax.lax.top_k(masked, _PRE)
    s = box_scores[idx]
    b = box_preds[idx]
    c = b[:, :3]
    d = jnp.abs(b[:, 3:6])
    mn = c - d * 0.5
    mx = c + d * 0.5
    vol = (d[:, 0] * d[:, 1] * d[:, 2])[:, None]
    cat = jnp.concatenate([mn, mx, vol], axis=1).T
    colaux = cat.reshape(7, _R, _C)
    rowaux = cat.reshape(7, _PRE, 1)
    valid = (s > _SCORE_THRESH).astype(jnp.float32).reshape(_R, _C)
    keep_f = pl.pallas_call(
        _nms_body,
        out_shape=jax.ShapeDtypeStruct((_R, _C), jnp.float32),
        scratch_shapes=[pltpu.VMEM((7, _C, _C), jnp.float32)],
    )(colaux, rowaux, valid)
    keep = keep_f.reshape(_PRE) > 0.5
    kept_masked = jnp.where(keep, s, -1.0)
    _, order = jax.lax.top_k(kept_masked, _POST)
    sel_valid = keep[order]
    sel_scores = jnp.where(sel_valid, s[order], 0.0)
    sel_idx = jnp.where(sel_valid, idx[order], -1)
    return sel_scores, sel_idx
